# Initial kernel scaffold; baseline (speedup 1.0000x reference)
import jax
import jax.numpy as jnp
from jax.experimental import pallas as pl


def kernel(inputs, feature_vector):
    raise NotImplementedError("write your pallas kernel here")


# speedup vs baseline: 523.4219x; 523.4219x over previous
import numpy as np
import jax
import jax.numpy as jnp
from jax import lax
from jax.experimental import pallas as pl

_BLK_A = 2000
_BLK_B = 10000
_N_BANK = 100000
_C = 64
_K = 9
_BIG = 3.0e38


def _resize_mat(inp=14, out=224):
    scale = inp / out
    x = (np.arange(out) + 0.5) * scale - 0.5
    j = np.arange(inp)
    w = np.maximum(0.0, 1.0 - np.abs(x[:, None] - j[None, :]))
    return w / w.sum(axis=1, keepdims=True)


def _blur_mat(n=224, sigma=4.0):
    r = int(round(4 * sigma))
    size = 2 * r + 1
    ax = np.arange(size) - r
    g = np.exp(-(ax * ax) / (2.0 * sigma * sigma))
    g = g / g.sum()
    G = np.zeros((n, n), np.float64)
    for i in range(n):
        lo = max(0, i - r)
        hi = min(n, i + r + 1)
        G[i, lo:hi] = g[(lo - i) + r:(hi - i) + r]
    return G


_A_MAT = jnp.asarray((_blur_mat() @ _resize_mat()).astype(np.float32))
_AT_MAT = _A_MAT.T


def _min_kern(at_ref, b_ref, o_ref):
    i = pl.program_id(0)
    b = b_ref[...]
    bb = jnp.sum(b * b, axis=1, keepdims=True)
    b_aug = jnp.concatenate([b, bb], axis=1)
    e = lax.dot_general(b_aug, at_ref[...], (((1,), (0,)), ((), ())),
                        preferred_element_type=jnp.float32)
    m = jnp.min(e, axis=0, keepdims=True)

    @pl.when(i == 0)
    def _():
        o_ref[...] = m

    @pl.when(i > 0)
    def _():
        o_ref[...] = jnp.minimum(o_ref[...], m)


def _topk_kern(a4_ref, aa4_ref, b_ref, top_ref, s_ref):
    i = pl.program_id(0)
    nb = pl.num_programs(0)

    @pl.when(i == 0)
    def _():
        top_ref[...] = jnp.full((8, 16), _BIG, jnp.float32)

    b = b_ref[...]
    bb = jnp.sum(b * b, axis=1, keepdims=True)
    b_aug = jnp.concatenate([b, bb], axis=1)
    e = lax.dot_general(a4_ref[...], b_aug, (((1,), (1,)), ((), ())),
                        preferred_element_type=jnp.float32)
    m = jnp.min(e, axis=1, keepdims=True)

    @pl.when(jnp.any(m < top_ref[:, _K - 1:_K]))
    def _():
        comb = jnp.concatenate([top_ref[...], e], axis=1)
        iota = lax.broadcasted_iota(jnp.int32, comb.shape, 1)
        cols = []
        for _ in range(_K):
            v = jnp.min(comb, axis=1, keepdims=True)
            am = jnp.argmin(comb, axis=1)
            comb = jnp.where(iota == am[:, None], _BIG, comb)
            cols.append(v)
        cols.append(jnp.full((8, 16 - _K), _BIG, jnp.float32))
        top_ref[...] = jnp.concatenate(cols, axis=1)

    @pl.when(i == nb - 1)
    def _():
        conf = jnp.sqrt(jnp.maximum(top_ref[:, :_K] + aa4_ref[...], 1e-12))
        ec = jnp.exp(conf)
        w = 1.0 - jnp.max(ec, axis=1, keepdims=True) / jnp.sum(ec, axis=1, keepdims=True)
        s_ref[...] = conf[:, 0:1] * w


def _mask_kern(d2_ref, a_ref, at_ref, o_ref):
    m14 = jnp.sqrt(jnp.maximum(d2_ref[0], 1e-12))
    t = jnp.dot(a_ref[...], m14, preferred_element_type=jnp.float32)
    o_ref[0] = jnp.dot(t, at_ref[...], preferred_element_type=jnp.float32)


def kernel(inputs, feature_vector):
    bsz, h, w, c = inputs.shape
    n_pix = bsz * h * w
    q = inputs.reshape(n_pix, c)
    aa = jnp.sum(q * q, axis=1)
    a_aug = jnp.concatenate([-2.0 * q, jnp.ones((n_pix, 1), jnp.float32)], axis=1)
    a_augT = a_aug.T

    min_e = pl.pallas_call(
        _min_kern,
        grid=(_N_BANK // _BLK_A,),
        in_specs=[
            pl.BlockSpec((c + 1, n_pix), lambda i: (0, 0)),
            pl.BlockSpec((_BLK_A, c), lambda i: (i, 0)),
        ],
        out_specs=pl.BlockSpec((1, n_pix), lambda i: (0, 0)),
        out_shape=jax.ShapeDtypeStruct((1, n_pix), jnp.float32),
    )(a_augT, feature_vector)

    d2min = aa + min_e[0]
    idx = jnp.argmax(d2min.reshape(bsz, h * w), axis=1)
    sel = idx + jnp.arange(bsz) * (h * w)
    a4 = jnp.concatenate([a_aug[sel], jnp.zeros((8 - bsz, c + 1), jnp.float32)], axis=0)
    aa4 = jnp.concatenate([aa[sel], jnp.zeros((8 - bsz,), jnp.float32)])[:, None]

    _, s8 = pl.pallas_call(
        _topk_kern,
        grid=(_N_BANK // _BLK_B,),
        in_specs=[
            pl.BlockSpec((8, c + 1), lambda i: (0, 0)),
            pl.BlockSpec((8, 1), lambda i: (0, 0)),
            pl.BlockSpec((_BLK_B, c), lambda i: (i, 0)),
        ],
        out_specs=[
            pl.BlockSpec((8, 16), lambda i: (0, 0)),
            pl.BlockSpec((8, 1), lambda i: (0, 0)),
        ],
        out_shape=[
            jax.ShapeDtypeStruct((8, 16), jnp.float32),
            jax.ShapeDtypeStruct((8, 1), jnp.float32),
        ],
    )(a4, aa4, feature_vector)
    s = s8[:bsz]

    mask = pl.pallas_call(
        _mask_kern,
        grid=(bsz,),
        in_specs=[
            pl.BlockSpec((1, h, w), lambda i: (i, 0, 0)),
            pl.BlockSpec((224, h), lambda i: (0, 0)),
            pl.BlockSpec((h, 224), lambda i: (0, 0)),
        ],
        out_specs=pl.BlockSpec((1, 224, 224), lambda i: (i, 0, 0)),
        out_shape=jax.ShapeDtypeStruct((bsz, 224, 224), jnp.float32),
    )(d2min.reshape(bsz, h, w), _A_MAT, _AT_MAT)

    return (s, mask.reshape(bsz, 224, 224, 1))
